# baseline (speedup 1.0000x reference)
import functools

import jax
import jax.numpy as jnp
from jax.experimental import pallas as pl
from jax.experimental.pallas import tpu as pltpu

_CT = 2048


def _kern(nc, q_ref, w_ref, b_ref, mem_ref, ret_ref, attn_ref, m_ref, s_ref):
    p = pl.program_id(0)
    c = pl.program_id(1)
    logits = jax.lax.dot_general(
        q_ref[:], w_ref[:], (((1,), (1,)), ((), ())),
        preferred_element_type=jnp.float32)
    logits = logits + b_ref[:]

    @pl.when(p == 0)
    def _phase0():
        tmax = jnp.max(logits, axis=1, keepdims=True)

        @pl.when(c == 0)
        def _():
            m_ref[:] = tmax
            s_ref[:] = jnp.sum(jnp.exp(logits - tmax), axis=1, keepdims=True)

        @pl.when(c > 0)
        def _():
            m_old = m_ref[:]
            m_new = jnp.maximum(m_old, tmax)
            s_ref[:] = (s_ref[:] * jnp.exp(m_old - m_new)
                        + jnp.sum(jnp.exp(logits - m_new), axis=1,
                                  keepdims=True))
            m_ref[:] = m_new

    @pl.when(p == 1)
    def _phase1():
        e = jnp.exp(logits - m_ref[:])
        attn_ref[:] = e * (1.0 / s_ref[:])
        contrib = jax.lax.dot_general(
            e, mem_ref[:], (((1,), (0,)), ((), ())),
            preferred_element_type=jnp.float32)

        @pl.when(c == 0)
        def _():
            ret_ref[:] = contrib

        @pl.when(c > 0)
        def _():
            ret_ref[:] = ret_ref[:] + contrib

        @pl.when(c == nc - 1)
        def _():
            ret_ref[:] = ret_ref[:] * (1.0 / s_ref[:])


def kernel(da_query, da_waaagh_memory, W_access, b_access):
    b_dim, d = da_query.shape
    cap = W_access.shape[0]
    nc = pl.cdiv(cap, _CT)
    cp = nc * _CT
    pad = cp - cap
    wp = jnp.pad(W_access, ((0, pad), (0, 0)))
    memp = jnp.pad(da_waaagh_memory, ((0, pad), (0, 0)))
    bp = jnp.pad(b_access.reshape(1, cap), ((0, 0), (0, pad)),
                 constant_values=-1e30)

    ret, attn = pl.pallas_call(
        functools.partial(_kern, nc),
        grid=(2, nc),
        in_specs=[
            pl.BlockSpec((b_dim, d), lambda p, c: (0, 0)),
            pl.BlockSpec((_CT, d), lambda p, c: (c, 0)),
            pl.BlockSpec((1, _CT), lambda p, c: (0, c)),
            pl.BlockSpec((_CT, d), lambda p, c: (p * c, 0)),
        ],
        out_specs=[
            pl.BlockSpec((b_dim, d), lambda p, c: (0, 0)),
            pl.BlockSpec((b_dim, _CT), lambda p, c: (0, p * c)),
        ],
        out_shape=[
            jax.ShapeDtypeStruct((b_dim, d), jnp.float32),
            jax.ShapeDtypeStruct((b_dim, cap), jnp.float32),
        ],
        scratch_shapes=[
            pltpu.VMEM((b_dim, 1), jnp.float32),
            pltpu.VMEM((b_dim, 1), jnp.float32),
        ],
    )(da_query, wp, bp, memp)
    return (ret, attn)
